# baseline (speedup 1.0000x reference)
import functools

import jax
import jax.numpy as jnp
from jax import lax
from jax.experimental import pallas as pl
from jax.experimental.pallas import tpu as pltpu
from jax.experimental.pallas import tpu_sc as plsc

B = 16384
A = 16
NC, NS = 2, 16
NW = NC * NS
BPW = B // NW
NBUF = 16
WAVES = BPW // NBUF
CHUNK = 128


def _sc_body(x_hbm, acT_hbm, qT_hbm, logitsT_hbm, value_hbm, qvT_hbm,
             idx_v, acb_v, qb_v, oac_v, oq_v, val_v, sem_ac, sem_q):
    wid = lax.axis_index("s") * NC + lax.axis_index("c")
    base = wid * BPW
    iota16 = lax.iota(jnp.int32, 16)

    pltpu.sync_copy(x_hbm.at[pl.ds(base, BPW)], idx_v)

    def start_fetch(r, buf):
        boff = pl.multiple_of((r // 128) * 128, 128)
        pltpu.make_async_copy(
            acT_hbm.at[:, pl.ds(boff, 128)], acb_v.at[buf], sem_ac).start()
        pltpu.make_async_copy(
            qT_hbm.at[:, pl.ds(boff, 128)], qb_v.at[buf], sem_q).start()

    def wait_fetch(buf):
        pltpu.make_async_copy(
            acT_hbm.at[:, pl.ds(0, 128)], acb_v.at[buf], sem_ac).wait()
        pltpu.make_async_copy(
            qT_hbm.at[:, pl.ds(0, 128)], qb_v.at[buf], sem_q).wait()

    rv0 = plsc.load_gather(idx_v, [iota16])
    for b in range(NBUF):
        start_fetch(rv0[b], b)

    def wave(w, carry):
        rv = plsc.load_gather(idx_v, [w * NBUF + iota16])
        wn = jnp.minimum(w + 1, WAVES - 1)
        rvn = plsc.load_gather(idx_v, [wn * NBUF + iota16])
        acc = jnp.zeros((16,), jnp.float32)
        for b in range(NBUF):
            r = rv[b]
            lane = jnp.full((16,), r % 128, jnp.int32)
            bsel = jnp.full((16,), b, jnp.int32)
            col = jnp.full((16,), (w % 8) * 16 + b, jnp.int32)
            wait_fetch(b)
            g_ac = plsc.load_gather(acb_v, [bsel, iota16, lane])
            g_q = plsc.load_gather(qb_v, [bsel, iota16, lane])

            @pl.when(w < WAVES - 1)
            def _():
                start_fetch(rvn[b], b)

            plsc.store_scatter(oac_v, [iota16, col], g_ac)
            plsc.store_scatter(oq_v, [iota16, col], g_q)
            m = jnp.max(g_q)
            acc = jnp.where(iota16 == b, m, acc)
        plsc.store_scatter(val_v, [w * NBUF + iota16], acc)

        @pl.when(w % 8 == 7)
        def _():
            coff = pl.multiple_of(base + (w // 8) * CHUNK, 128)
            pltpu.sync_copy(oac_v, logitsT_hbm.at[:, pl.ds(coff, CHUNK)])
            pltpu.sync_copy(oq_v, qvT_hbm.at[:, pl.ds(coff, CHUNK)])

        return carry

    lax.fori_loop(0, WAVES, wave, 0)
    pltpu.sync_copy(val_v, value_hbm.at[pl.ds(base, BPW)])


@jax.jit
def _run(x, ac_table, q_table):
    mesh = plsc.VectorSubcoreMesh(core_axis_name="c", subcore_axis_name="s")
    out_type = (
        jax.ShapeDtypeStruct((A, B), jnp.float32),
        jax.ShapeDtypeStruct((B,), jnp.float32),
        jax.ShapeDtypeStruct((A, B), jnp.float32),
    )
    scratch = [
        pltpu.VMEM((BPW,), jnp.int32),
        pltpu.VMEM((NBUF, A, 128), jnp.float32),
        pltpu.VMEM((NBUF, A, 128), jnp.float32),
        pltpu.VMEM((A, CHUNK), jnp.float32),
        pltpu.VMEM((A, CHUNK), jnp.float32),
        pltpu.VMEM((BPW,), jnp.float32),
        pltpu.SemaphoreType.DMA,
        pltpu.SemaphoreType.DMA,
    ]
    k = pl.kernel(_sc_body, out_type=out_type, mesh=mesh,
                  scratch_types=scratch,
                  compiler_params=pltpu.CompilerParams(
                      needs_layout_passes=False,
                      use_tc_tiling_on_sc=True))
    lT, v, qT = k(x, ac_table.T, q_table.T)
    return lT.T, v, qT.T


def kernel(x, ac_table, q_table):
    return _run(x, ac_table, q_table)
